# baseline (speedup 1.0000x reference)
import functools

import jax
import jax.numpy as jnp
from jax import lax
from jax.experimental import pallas as pl
from jax.experimental.pallas import tpu as pltpu
from jax.experimental.pallas import tpu_sc as plsc

_SEQ = 200
_LAG = 50
_D = 64
_B = 1024
_NC = 2
_NS = 16
_NW = _NC * _NS
_BPW = _B // _NW
_ICH = 100
_NCH = _SEQ // _ICH


def _body(xidx_hbm, w0, w1, w2, w3, w4, cidx_hbm, out_hbm,
          xall, cidx_v, rows0, rows1, c345, sem_g, sem_w0, sem_w1):
  cid = lax.axis_index("c")
  sid = lax.axis_index("s")
  wid = sid * _NC + cid
  b0 = wid * _BPW

  pltpu.sync_copy(xidx_hbm.at[pl.ds(b0, _BPW)], xall)

  pltpu.sync_copy(cidx_hbm, cidx_v)
  pltpu.sync_copy(w2, c345.at[:, 0, :])
  for t, w in ((0, w3), (1, w4)):
    tmp = (rows0, rows1)[t].at[0]
    for k in range(_NCH):
      pltpu.async_copy(w.at[cidx_v.at[t, k]],
                       tmp.at[pl.ds(k * _ICH, _ICH)], sem_g)
    pltpu.make_async_copy(w.at[cidx_v.at[t, 0]], tmp, sem_g).wait()

  def pack_row(s, carry):
    for t in range(2):
      tmp = (rows0, rows1)[t].at[0]
      for k in range(_D // 16):
        c345[s, t + 1, pl.ds(16 * k, 16)] = tmp[s, pl.ds(16 * k, 16)]
    return carry

  lax.fori_loop(0, _SEQ, pack_row, 0)

  sem_w = (sem_w0, sem_w1)

  def fire_g(i, p):
    for k in range(_NCH):
      pltpu.async_copy(w0.at[xall.at[i, 0, k]],
                       rows0.at[p, pl.ds(k * _ICH, _ICH)], sem_g)
      pltpu.async_copy(w1.at[xall.at[i, 1, k]],
                       rows1.at[p, pl.ds(k * _ICH, _ICH)], sem_g)

  def wait_g(i, p):
    pltpu.make_async_copy(w0.at[xall.at[i, 0, 0]], rows0.at[p], sem_g).wait()
    pltpu.make_async_copy(w1.at[xall.at[i, 1, 0]], rows1.at[p], sem_g).wait()

  def fire_w(i, p):
    b = b0 + i
    pltpu.async_copy(rows0.at[p], out_hbm.at[b, :, 0, :], sem_w[p])
    pltpu.async_copy(rows1.at[p], out_hbm.at[b, :, 1, :], sem_w[p])
    pltpu.async_copy(c345, out_hbm.at[b, :, pl.ds(2, 3), :], sem_w[p])

  def wait_w(i, p):
    b = b0 + i
    pltpu.make_async_copy(rows0.at[p], out_hbm.at[b, :, 0, :], sem_w[p]).wait()
    pltpu.make_async_copy(rows1.at[p], out_hbm.at[b, :, 1, :], sem_w[p]).wait()
    pltpu.make_async_copy(c345, out_hbm.at[b, :, pl.ds(2, 3), :],
                          sem_w[p]).wait()

  def pair(j, first, last):
    a = 2 * j
    b = 2 * j + 1
    wait_g(a, 0)
    fire_w(a, 0)
    if not first:
      wait_w(b - 2, 1)
    fire_g(b, 1)
    wait_g(b, 1)
    fire_w(b, 1)
    wait_w(a, 0)
    if not last:
      fire_g(b + 1, 0)

  fire_g(0, 0)
  pair(0, True, False)
  lax.fori_loop(1, _BPW // 2 - 1, lambda j, c: (pair(j, False, False), c)[1], 0)
  pair(_BPW // 2 - 1, False, True)
  wait_w(_BPW - 1, 1)


def kernel(x, W0, W1, W2, W3, W4):
  xidx = x.astype(jnp.int32).transpose(0, 2, 1).reshape(_B, 2, _NCH, _ICH)
  pf = jnp.concatenate([jnp.zeros(_SEQ - _LAG, jnp.int32),
                        jnp.arange(1, _LAG + 1, dtype=jnp.int32)])
  isf = (jnp.arange(_SEQ, dtype=jnp.int32) >= (_SEQ - _LAG)).astype(jnp.int32)
  cidx = jnp.stack([pf, isf]).reshape(2, _NCH, _ICH)

  mesh = plsc.VectorSubcoreMesh(core_axis_name="c", subcore_axis_name="s")
  run = pl.kernel(
      _body,
      out_type=jax.ShapeDtypeStruct((_B, _SEQ, 5, _D), jnp.float32),
      mesh=mesh,
      scratch_types=[
          pltpu.VMEM((_BPW, 2, _NCH, _ICH), jnp.int32),
          pltpu.VMEM((2, _NCH, _ICH), jnp.int32),
          pltpu.VMEM((2, _SEQ, _D), jnp.float32),
          pltpu.VMEM((2, _SEQ, _D), jnp.float32),
          pltpu.VMEM((_SEQ, 3, _D), jnp.float32),
          pltpu.SemaphoreType.DMA,
          pltpu.SemaphoreType.DMA,
          pltpu.SemaphoreType.DMA,
      ],
      compiler_params=pltpu.CompilerParams(use_tc_tiling_on_sc=False),
  )
  return run(xidx, W0, W1, W2, W3, W4, cidx)
